# baseline (speedup 1.0000x reference)
import functools

import jax
import jax.numpy as jnp
from jax import lax
from jax.experimental import pallas as pl
from jax.experimental.pallas import tpu as pltpu
from jax.experimental.pallas import tpu_sc as plsc

_M = 100000
_MP = 100352
_B = 16384
_D = 128
_NT = 16
_NC = 2
_ROUNDS = _D // (_NT * _NC)
_L = 16
_CH = 8192
_H0, _H1 = 49152, _MP - 49152


def _make_sc_scatter():
    mesh = plsc.VectorSubcoreMesh(core_axis_name="c", subcore_axis_name="s")

    @functools.partial(
        pl.kernel,
        mesh=mesh,
        out_type=jax.ShapeDtypeStruct((_D * _MP,), jnp.float32),
        scratch_types=[
            pltpu.VMEM_SHARED((_NT * _MP,), jnp.float32),
            pltpu.VMEM((_CH,), jnp.int32),
            pltpu.VMEM((_CH,), jnp.float32),
        ],
    )
    def sc_scatter(xt_hbm, idxt_hbm, srct_hbm, outt_hbm, acc_sh, idx_v, src_v):
        c = lax.axis_index("c")
        s = lax.axis_index("s")
        base = s * _MP

        def round_body(r, carry):
            col = c * (_ROUNDS * _NT) + r * _NT + s
            for off, ln in ((0, _H0), (_H0, _H1)):
                pltpu.sync_copy(xt_hbm.at[pl.ds(col * _MP + off, ln)],
                                acc_sh.at[pl.ds(base + off, ln)])

            def chunk_body(h, carry2):
                cbase = col * _B + h * _CH
                pltpu.sync_copy(idxt_hbm.at[pl.ds(cbase, _CH)], idx_v)
                pltpu.sync_copy(srct_hbm.at[pl.ds(cbase, _CH)], src_v)

                def off_body(i, carry3):
                    idx_v[pl.ds(i * _L, _L)] = idx_v[pl.ds(i * _L, _L)] + base
                    return carry3

                lax.fori_loop(0, _CH // _L, off_body, 0, unroll=4)
                pltpu.sync_copy(src_v, acc_sh.at[idx_v], add=True)
                return carry2

            lax.fori_loop(0, _B // _CH, chunk_body, 0)
            for off, ln in ((0, _H0), (_H0, _H1)):
                pltpu.sync_copy(acc_sh.at[pl.ds(base + off, ln)],
                                outt_hbm.at[pl.ds(col * _MP + off, ln)])
            return carry

        lax.fori_loop(0, _ROUNDS, round_body, 0)

    return sc_scatter


def _tc_transpose_x(x):
    br = 1024

    def body(x_ref, o_ref):
        o_ref[...] = x_ref[...].T.reshape(_D, br // _D, _D)

    return pl.pallas_call(
        body,
        grid=(_MP // br,),
        in_specs=[pl.BlockSpec((br, _D), lambda j: (j, 0))],
        out_specs=pl.BlockSpec((_D, br // _D, _D), lambda j: (0, j, 0)),
        out_shape=jax.ShapeDtypeStruct((_D, _MP // _D, _D), jnp.float32),
    )(x)


def _tc_transpose_src_idx(src, idx):
    br = 2048

    def body(s_ref, i_ref, os_ref, oi_ref):
        os_ref[...] = s_ref[...].T.reshape(_D, br // _D, _D)
        oi_ref[...] = i_ref[...].T.reshape(_D, br // _D, _D)

    return pl.pallas_call(
        body,
        grid=(_B // br,),
        in_specs=[pl.BlockSpec((br, _D), lambda j: (j, 0)),
                  pl.BlockSpec((br, _D), lambda j: (j, 0))],
        out_specs=[pl.BlockSpec((_D, br // _D, _D), lambda j: (0, j, 0)),
                   pl.BlockSpec((_D, br // _D, _D), lambda j: (0, j, 0))],
        out_shape=[jax.ShapeDtypeStruct((_D, _B // _D, _D), jnp.float32),
                   jax.ShapeDtypeStruct((_D, _B // _D, _D), jnp.int32)],
    )(src, idx)


def _tc_transpose_out(outt):
    br = 1024

    def body(t_ref, o_ref):
        o_ref[...] = t_ref[...].reshape(_D, br).T

    return pl.pallas_call(
        body,
        grid=(_MP // br,),
        in_specs=[pl.BlockSpec((_D, br // _D, _D), lambda j: (0, j, 0))],
        out_specs=pl.BlockSpec((br, _D), lambda j: (j, 0)),
        out_shape=jax.ShapeDtypeStruct((_M, _D), jnp.float32),
    )(outt)


def kernel(x, dim, index, src, include_self):
    xt = jnp.pad(x, ((0, _MP - _M), (0, 0))).T.reshape(-1)
    srct = src.T.reshape(-1)
    idxt = index.astype(jnp.int32).T.reshape(-1)
    outt = _make_sc_scatter()(xt, idxt, srct)
    return outt.reshape(_D, _MP)[:, :_M].T


# speedup vs baseline: 1.2600x; 1.2600x over previous
import functools

import jax
import jax.numpy as jnp
from jax import lax
from jax.experimental import pallas as pl
from jax.experimental.pallas import tpu as pltpu
from jax.experimental.pallas import tpu_sc as plsc

_M = 100000
_MP = 100352
_B = 16384
_D = 128
_NT = 16
_NC = 2
_ROUNDS = _D // (_NT * _NC)
_L = 16
_CH = 8192
_H0, _H1 = 49152, _MP - 49152


def _make_sc_scatter():
    mesh = plsc.VectorSubcoreMesh(core_axis_name="c", subcore_axis_name="s")

    @functools.partial(
        pl.kernel,
        mesh=mesh,
        out_type=jax.ShapeDtypeStruct((_D * _MP,), jnp.float32),
        scratch_types=[
            pltpu.VMEM_SHARED((_NT * _MP,), jnp.float32),
            pltpu.VMEM((_CH,), jnp.int32),
            pltpu.VMEM((_CH,), jnp.float32),
        ],
    )
    def sc_scatter(xt_hbm, idxt_hbm, srct_hbm, outt_hbm, acc_sh, idx_v, src_v):
        c = lax.axis_index("c")
        s = lax.axis_index("s")
        base = s * _MP

        def round_body(r, carry):
            col = c * (_ROUNDS * _NT) + r * _NT + s
            for off, ln in ((0, _H0), (_H0, _H1)):
                pltpu.sync_copy(xt_hbm.at[pl.ds(col * _MP + off, ln)],
                                acc_sh.at[pl.ds(base + off, ln)])

            def chunk_body(h, carry2):
                cbase = col * _B + h * _CH
                pltpu.sync_copy(idxt_hbm.at[pl.ds(cbase, _CH)], idx_v)
                pltpu.sync_copy(srct_hbm.at[pl.ds(cbase, _CH)], src_v)

                def off_body(i, carry3):
                    idx_v[pl.ds(i * _L, _L)] = idx_v[pl.ds(i * _L, _L)] + base
                    return carry3

                lax.fori_loop(0, _CH // _L, off_body, 0, unroll=4)
                pltpu.sync_copy(src_v, acc_sh.at[idx_v], add=True)
                return carry2

            lax.fori_loop(0, _B // _CH, chunk_body, 0)
            for off, ln in ((0, _H0), (_H0, _H1)):
                pltpu.sync_copy(acc_sh.at[pl.ds(base + off, ln)],
                                outt_hbm.at[pl.ds(col * _MP + off, ln)])
            return carry

        lax.fori_loop(0, _ROUNDS, round_body, 0)

    return sc_scatter


def _tc_transpose_x(x):
    br = 1024

    def body(x_ref, o_ref):
        o_ref[...] = x_ref[...].T.reshape(_D, br // _D, _D)

    return pl.pallas_call(
        body,
        grid=(_MP // br,),
        in_specs=[pl.BlockSpec((br, _D), lambda j: (j, 0))],
        out_specs=pl.BlockSpec((_D, br // _D, _D), lambda j: (0, j, 0)),
        out_shape=jax.ShapeDtypeStruct((_D, _MP // _D, _D), jnp.float32),
    )(x)


def _tc_transpose_src_idx(src, idx):
    br = 2048

    def body(s_ref, i_ref, os_ref, oi_ref):
        os_ref[...] = s_ref[...].T.reshape(_D, br // _D, _D)
        oi_ref[...] = i_ref[...].T.reshape(_D, br // _D, _D)

    return pl.pallas_call(
        body,
        grid=(_B // br,),
        in_specs=[pl.BlockSpec((br, _D), lambda j: (j, 0)),
                  pl.BlockSpec((br, _D), lambda j: (j, 0))],
        out_specs=[pl.BlockSpec((_D, br // _D, _D), lambda j: (0, j, 0)),
                   pl.BlockSpec((_D, br // _D, _D), lambda j: (0, j, 0))],
        out_shape=[jax.ShapeDtypeStruct((_D, _B // _D, _D), jnp.float32),
                   jax.ShapeDtypeStruct((_D, _B // _D, _D), jnp.int32)],
    )(src, idx)


def _tc_transpose_out(outt):
    br = 1024

    def body(t_ref, o_ref):
        o_ref[...] = t_ref[...].reshape(_D, br).T

    return pl.pallas_call(
        body,
        grid=(_MP // br,),
        in_specs=[pl.BlockSpec((_D, br // _D, _D), lambda j: (0, j, 0))],
        out_specs=pl.BlockSpec((br, _D), lambda j: (j, 0)),
        out_shape=jax.ShapeDtypeStruct((_M, _D), jnp.float32),
    )(outt)


def kernel(x, dim, index, src, include_self):
    xt = _tc_transpose_x(x).reshape(-1)
    srct = src.T.reshape(-1)
    idxt = index.astype(jnp.int32).T.reshape(-1)
    outt = _make_sc_scatter()(xt, idxt, srct)
    return _tc_transpose_out(outt.reshape(_D, _MP // _D, _D))


# speedup vs baseline: 1.5952x; 1.2661x over previous
import functools

import jax
import jax.numpy as jnp
from jax import lax
from jax.experimental import pallas as pl
from jax.experimental.pallas import tpu as pltpu
from jax.experimental.pallas import tpu_sc as plsc

_M = 100000
_MP = 100352
_B = 16384
_D = 128
_NT = 16
_NC = 2
_ROUNDS = _D // (_NT * _NC)
_L = 16
_CH = 8192
_H0, _H1 = 49152, _MP - 49152


def _make_sc_scatter():
    mesh = plsc.VectorSubcoreMesh(core_axis_name="c", subcore_axis_name="s")

    @functools.partial(
        pl.kernel,
        mesh=mesh,
        out_type=jax.ShapeDtypeStruct((_D * _MP,), jnp.float32),
        scratch_types=[
            pltpu.VMEM_SHARED((_NT * _MP,), jnp.float32),
            pltpu.VMEM((_CH,), jnp.int32),
            pltpu.VMEM((_CH,), jnp.float32),
        ],
    )
    def sc_scatter(xt_hbm, idxt_hbm, srct_hbm, outt_hbm, acc_sh, idx_v, src_v):
        c = lax.axis_index("c")
        s = lax.axis_index("s")
        base = s * _MP

        def round_body(r, carry):
            col = c * (_ROUNDS * _NT) + r * _NT + s
            for off, ln in ((0, _H0), (_H0, _H1)):
                pltpu.sync_copy(xt_hbm.at[pl.ds(col * _MP + off, ln)],
                                acc_sh.at[pl.ds(base + off, ln)])

            def chunk_body(h, carry2):
                cbase = col * _B + h * _CH
                pltpu.sync_copy(idxt_hbm.at[pl.ds(cbase, _CH)], idx_v)
                pltpu.sync_copy(srct_hbm.at[pl.ds(cbase, _CH)], src_v)

                def off_body(i, carry3):
                    idx_v[pl.ds(i * _L, _L)] = idx_v[pl.ds(i * _L, _L)] + base
                    return carry3

                lax.fori_loop(0, _CH // _L, off_body, 0, unroll=4)
                pltpu.sync_copy(src_v, acc_sh.at[idx_v], add=True)
                return carry2

            lax.fori_loop(0, _B // _CH, chunk_body, 0)
            for off, ln in ((0, _H0), (_H0, _H1)):
                pltpu.sync_copy(acc_sh.at[pl.ds(base + off, ln)],
                                outt_hbm.at[pl.ds(col * _MP + off, ln)])
            return carry

        lax.fori_loop(0, _ROUNDS, round_body, 0)

    return sc_scatter


def _tc_transpose_x(x):
    br = 2048

    def body(x_ref, o_ref):
        o_ref[...] = x_ref[...].T.reshape(_D, br // _D, _D)

    return pl.pallas_call(
        body,
        grid=(_MP // br,),
        in_specs=[pl.BlockSpec((br, _D), lambda j: (j, 0))],
        out_specs=pl.BlockSpec((_D, br // _D, _D), lambda j: (0, j, 0)),
        out_shape=jax.ShapeDtypeStruct((_D, _MP // _D, _D), jnp.float32),
    )(x)


def _tc_transpose_src_idx(src, idx):
    br = 2048

    def body(s_ref, i_ref, os_ref, oi_ref):
        os_ref[...] = s_ref[...].T.reshape(_D, br // _D, _D)
        oi_ref[...] = i_ref[...].T.reshape(_D, br // _D, _D)

    return pl.pallas_call(
        body,
        grid=(_B // br,),
        in_specs=[pl.BlockSpec((br, _D), lambda j: (j, 0)),
                  pl.BlockSpec((br, _D), lambda j: (j, 0))],
        out_specs=[pl.BlockSpec((_D, br // _D, _D), lambda j: (0, j, 0)),
                   pl.BlockSpec((_D, br // _D, _D), lambda j: (0, j, 0))],
        out_shape=[jax.ShapeDtypeStruct((_D, _B // _D, _D), jnp.float32),
                   jax.ShapeDtypeStruct((_D, _B // _D, _D), jnp.int32)],
    )(src, idx)


def _tc_transpose_out(outt):
    br = 2048

    def body(t_ref, o_ref):
        o_ref[...] = t_ref[...].reshape(_D, br).T

    return pl.pallas_call(
        body,
        grid=(_MP // br,),
        in_specs=[pl.BlockSpec((_D, br // _D, _D), lambda j: (0, j, 0))],
        out_specs=pl.BlockSpec((br, _D), lambda j: (j, 0)),
        out_shape=jax.ShapeDtypeStruct((_M, _D), jnp.float32),
    )(outt)


def kernel(x, dim, index, src, include_self):
    xt = _tc_transpose_x(x).reshape(-1)
    srct, idxt = _tc_transpose_src_idx(src, index.astype(jnp.int32))
    outt = _make_sc_scatter()(xt, idxt.reshape(-1), srct.reshape(-1))
    return _tc_transpose_out(outt.reshape(_D, _MP // _D, _D))


# speedup vs baseline: 1.9794x; 1.2408x over previous
import functools

import jax
import jax.numpy as jnp
from jax import lax
from jax.experimental import pallas as pl
from jax.experimental.pallas import tpu as pltpu
from jax.experimental.pallas import tpu_sc as plsc

_M = 100000
_MP = 100352
_B = 16384
_D = 128
_NT = 16
_NC = 2
_ROUNDS = _D // (_NT * _NC)
_L = 16
_CH = 8192
_H0, _H1 = 49152, _MP - 49152


def _make_sc_scatter():
    mesh = plsc.VectorSubcoreMesh(core_axis_name="c", subcore_axis_name="s")

    @functools.partial(
        pl.kernel,
        mesh=mesh,
        out_type=jax.ShapeDtypeStruct((_D * _MP,), jnp.float32),
        scratch_types=[
            pltpu.VMEM_SHARED((_NT * _MP,), jnp.float32),
            pltpu.VMEM((_CH,), jnp.int32),
            pltpu.VMEM((_CH,), jnp.float32),
        ],
    )
    def sc_scatter(xt_hbm, idxt_hbm, srct_hbm, outt_hbm, acc_sh, idx_v, src_v):
        c = lax.axis_index("c")
        s = lax.axis_index("s")
        base = s * _MP

        def round_body(r, carry):
            col = c * (_ROUNDS * _NT) + r * _NT + s
            for off, ln in ((0, _H0), (_H0, _H1)):
                pltpu.sync_copy(xt_hbm.at[pl.ds(col * _MP + off, ln)],
                                acc_sh.at[pl.ds(base + off, ln)])

            def chunk_body(h, carry2):
                cbase = col * _B + h * _CH
                pltpu.sync_copy(idxt_hbm.at[pl.ds(cbase, _CH)], idx_v)
                pltpu.sync_copy(srct_hbm.at[pl.ds(cbase, _CH)], src_v)

                def off_body(i, carry3):
                    idx_v[pl.ds(i * _L, _L)] = idx_v[pl.ds(i * _L, _L)] + base
                    return carry3

                lax.fori_loop(0, _CH // _L, off_body, 0, unroll=4)
                pltpu.sync_copy(src_v, acc_sh.at[idx_v], add=True)
                return carry2

            lax.fori_loop(0, _B // _CH, chunk_body, 0)
            for off, ln in ((0, _H0), (_H0, _H1)):
                pltpu.sync_copy(acc_sh.at[pl.ds(base + off, ln)],
                                outt_hbm.at[pl.ds(col * _MP + off, ln)])
            return carry

        lax.fori_loop(0, _ROUNDS, round_body, 0)

    return sc_scatter


def _tc_transpose_x(x):
    br = 7168

    def body(x_ref, o_ref):
        o_ref[...] = x_ref[...].T.reshape(_D, br // _D, _D)

    return pl.pallas_call(
        body,
        grid=(_MP // br,),
        in_specs=[pl.BlockSpec((br, _D), lambda j: (j, 0))],
        out_specs=pl.BlockSpec((_D, br // _D, _D), lambda j: (0, j, 0)),
        out_shape=jax.ShapeDtypeStruct((_D, _MP // _D, _D), jnp.float32),
    )(x)


def _tc_transpose_src_idx(src, idx):
    br = 4096

    def body(s_ref, i_ref, os_ref, oi_ref):
        os_ref[...] = s_ref[...].T.reshape(_D, br // _D, _D)
        oi_ref[...] = i_ref[...].T.reshape(_D, br // _D, _D)

    return pl.pallas_call(
        body,
        grid=(_B // br,),
        in_specs=[pl.BlockSpec((br, _D), lambda j: (j, 0)),
                  pl.BlockSpec((br, _D), lambda j: (j, 0))],
        out_specs=[pl.BlockSpec((_D, br // _D, _D), lambda j: (0, j, 0)),
                   pl.BlockSpec((_D, br // _D, _D), lambda j: (0, j, 0))],
        out_shape=[jax.ShapeDtypeStruct((_D, _B // _D, _D), jnp.float32),
                   jax.ShapeDtypeStruct((_D, _B // _D, _D), jnp.int32)],
    )(src, idx)


def _tc_transpose_out(outt):
    br = 7168

    def body(t_ref, o_ref):
        o_ref[...] = t_ref[...].reshape(_D, br).T

    return pl.pallas_call(
        body,
        grid=(_MP // br,),
        in_specs=[pl.BlockSpec((_D, br // _D, _D), lambda j: (0, j, 0))],
        out_specs=pl.BlockSpec((br, _D), lambda j: (j, 0)),
        out_shape=jax.ShapeDtypeStruct((_M, _D), jnp.float32),
    )(outt)


def kernel(x, dim, index, src, include_self):
    xt = _tc_transpose_x(x).reshape(-1)
    srct, idxt = _tc_transpose_src_idx(src, index.astype(jnp.int32))
    outt = _make_sc_scatter()(xt, idxt.reshape(-1), srct.reshape(-1))
    return _tc_transpose_out(outt.reshape(_D, _MP // _D, _D))


# speedup vs baseline: 2.0189x; 1.0200x over previous
import functools

import jax
import jax.numpy as jnp
from jax import lax
from jax.experimental import pallas as pl
from jax.experimental.pallas import tpu as pltpu
from jax.experimental.pallas import tpu_sc as plsc

_M = 100000
_MP = 100352
_B = 16384
_D = 128
_NT = 16
_NC = 2
_ROUNDS = _D // (_NT * _NC)
_L = 16
_CH = 8192
_H0, _H1 = 49152, _MP - 49152


def _make_sc_scatter():
    mesh = plsc.VectorSubcoreMesh(core_axis_name="c", subcore_axis_name="s")

    @functools.partial(
        pl.kernel,
        mesh=mesh,
        out_type=jax.ShapeDtypeStruct((_D * _MP,), jnp.float32),
        scratch_types=[
            pltpu.VMEM_SHARED((_NT * _MP,), jnp.float32),
            pltpu.VMEM((_CH,), jnp.int32),
            pltpu.VMEM((_CH,), jnp.float32),
        ],
    )
    def sc_scatter(xt_hbm, idxt_hbm, srct_hbm, outt_hbm, acc_sh, idx_v, src_v):
        c = lax.axis_index("c")
        s = lax.axis_index("s")
        base = s * _MP

        def round_body(r, carry):
            col = c * (_ROUNDS * _NT) + r * _NT + s
            for off, ln in ((0, _H0), (_H0, _H1)):
                pltpu.sync_copy(xt_hbm.at[pl.ds(col * _MP + off, ln)],
                                acc_sh.at[pl.ds(base + off, ln)])

            def chunk_body(h, carry2):
                cbase = col * _B + h * _CH
                pltpu.sync_copy(idxt_hbm.at[pl.ds(cbase, _CH)], idx_v)
                pltpu.sync_copy(srct_hbm.at[pl.ds(cbase, _CH)], src_v)

                def off_body(i, carry3):
                    idx_v[pl.ds(i * _L, _L)] = idx_v[pl.ds(i * _L, _L)] + base
                    return carry3

                lax.fori_loop(0, _CH // _L, off_body, 0, unroll=4)
                pltpu.sync_copy(src_v, acc_sh.at[idx_v], add=True)
                return carry2

            lax.fori_loop(0, _B // _CH, chunk_body, 0)
            for off, ln in ((0, _H0), (_H0, _H1)):
                pltpu.sync_copy(acc_sh.at[pl.ds(base + off, ln)],
                                outt_hbm.at[pl.ds(col * _MP + off, ln)])
            return carry

        lax.fori_loop(0, _ROUNDS, round_body, 0)

    return sc_scatter


def _tc_transpose_x(x):
    br = 14336

    def body(x_ref, o_ref):
        o_ref[...] = x_ref[...].T.reshape(_D, br // _D, _D)

    return pl.pallas_call(
        body,
        grid=(_MP // br,),
        in_specs=[pl.BlockSpec((br, _D), lambda j: (j, 0))],
        out_specs=pl.BlockSpec((_D, br // _D, _D), lambda j: (0, j, 0)),
        out_shape=jax.ShapeDtypeStruct((_D, _MP // _D, _D), jnp.float32),
    )(x)


def _tc_transpose_src_idx(src, idx):
    br = 8192

    def body(s_ref, i_ref, os_ref, oi_ref):
        os_ref[...] = s_ref[...].T.reshape(_D, br // _D, _D)
        oi_ref[...] = i_ref[...].T.reshape(_D, br // _D, _D)

    return pl.pallas_call(
        body,
        grid=(_B // br,),
        in_specs=[pl.BlockSpec((br, _D), lambda j: (j, 0)),
                  pl.BlockSpec((br, _D), lambda j: (j, 0))],
        out_specs=[pl.BlockSpec((_D, br // _D, _D), lambda j: (0, j, 0)),
                   pl.BlockSpec((_D, br // _D, _D), lambda j: (0, j, 0))],
        out_shape=[jax.ShapeDtypeStruct((_D, _B // _D, _D), jnp.float32),
                   jax.ShapeDtypeStruct((_D, _B // _D, _D), jnp.int32)],
    )(src, idx)


def _tc_transpose_out(outt):
    br = 14336

    def body(t_ref, o_ref):
        o_ref[...] = t_ref[...].reshape(_D, br).T

    return pl.pallas_call(
        body,
        grid=(_MP // br,),
        in_specs=[pl.BlockSpec((_D, br // _D, _D), lambda j: (0, j, 0))],
        out_specs=pl.BlockSpec((br, _D), lambda j: (j, 0)),
        out_shape=jax.ShapeDtypeStruct((_M, _D), jnp.float32),
    )(outt)


def kernel(x, dim, index, src, include_self):
    xt = _tc_transpose_x(x).reshape(-1)
    srct, idxt = _tc_transpose_src_idx(src, index.astype(jnp.int32))
    outt = _make_sc_scatter()(xt, idxt.reshape(-1), srct.reshape(-1))
    return _tc_transpose_out(outt.reshape(_D, _MP // _D, _D))


# speedup vs baseline: 2.0634x; 1.0221x over previous
import functools

import jax
import jax.numpy as jnp
from jax import lax
from jax.experimental import pallas as pl
from jax.experimental.pallas import tpu as pltpu
from jax.experimental.pallas import tpu_sc as plsc

_M = 100000
_MP = 100352
_B = 16384
_D = 128
_NT = 16
_NC = 2
_ROUNDS = _D // (_NT * _NC)
_L = 16
_CH = 8192
_H0, _H1 = 49152, _MP - 49152


def _make_sc_scatter():
    mesh = plsc.VectorSubcoreMesh(core_axis_name="c", subcore_axis_name="s")

    @functools.partial(
        pl.kernel,
        mesh=mesh,
        out_type=jax.ShapeDtypeStruct((_D * _MP,), jnp.float32),
        scratch_types=[
            pltpu.VMEM_SHARED((_NT * _MP,), jnp.float32),
            pltpu.VMEM((_CH,), jnp.int32),
            pltpu.VMEM((_CH,), jnp.float32),
        ],
    )
    def sc_scatter(xt_hbm, idxt_hbm, srct_hbm, outt_hbm, acc_sh, idx_v, src_v):
        c = lax.axis_index("c")
        s = lax.axis_index("s")
        base = s * _MP

        def round_body(r, carry):
            col = c * (_ROUNDS * _NT) + r * _NT + s
            for off, ln in ((0, _H0), (_H0, _H1)):
                pltpu.sync_copy(xt_hbm.at[pl.ds(col * _MP + off, ln)],
                                acc_sh.at[pl.ds(base + off, ln)])

            def chunk_body(h, carry2):
                cbase = col * _B + h * _CH
                pltpu.sync_copy(idxt_hbm.at[pl.ds(cbase, _CH)], idx_v)
                pltpu.sync_copy(srct_hbm.at[pl.ds(cbase, _CH)], src_v)
                pltpu.sync_copy(src_v, acc_sh.at[idx_v], add=True)
                return carry2

            lax.fori_loop(0, _B // _CH, chunk_body, 0)
            for off, ln in ((0, _H0), (_H0, _H1)):
                pltpu.sync_copy(acc_sh.at[pl.ds(base + off, ln)],
                                outt_hbm.at[pl.ds(col * _MP + off, ln)])
            return carry

        lax.fori_loop(0, _ROUNDS, round_body, 0)

    return sc_scatter


def _tc_transpose_x(x):
    br = 14336

    def body(x_ref, o_ref):
        o_ref[...] = x_ref[...].T.reshape(_D, br // _D, _D)

    return pl.pallas_call(
        body,
        grid=(_MP // br,),
        in_specs=[pl.BlockSpec((br, _D), lambda j: (j, 0))],
        out_specs=pl.BlockSpec((_D, br // _D, _D), lambda j: (0, j, 0)),
        out_shape=jax.ShapeDtypeStruct((_D, _MP // _D, _D), jnp.float32),
    )(x)


def _tc_transpose_src_idx(src, idx):
    br = 8192

    def body(s_ref, i_ref, os_ref, oi_ref):
        cmap = (lax.broadcasted_iota(jnp.int32, (_D, 1), 0) % _NT) * _MP
        os_ref[...] = s_ref[...].T.reshape(_D, br // _D, _D)
        oi_ref[...] = (i_ref[...].T + cmap).reshape(_D, br // _D, _D)

    return pl.pallas_call(
        body,
        grid=(_B // br,),
        in_specs=[pl.BlockSpec((br, _D), lambda j: (j, 0)),
                  pl.BlockSpec((br, _D), lambda j: (j, 0))],
        out_specs=[pl.BlockSpec((_D, br // _D, _D), lambda j: (0, j, 0)),
                   pl.BlockSpec((_D, br // _D, _D), lambda j: (0, j, 0))],
        out_shape=[jax.ShapeDtypeStruct((_D, _B // _D, _D), jnp.float32),
                   jax.ShapeDtypeStruct((_D, _B // _D, _D), jnp.int32)],
    )(src, idx)


def _tc_transpose_out(outt):
    br = 14336

    def body(t_ref, o_ref):
        o_ref[...] = t_ref[...].reshape(_D, br).T

    return pl.pallas_call(
        body,
        grid=(_MP // br,),
        in_specs=[pl.BlockSpec((_D, br // _D, _D), lambda j: (0, j, 0))],
        out_specs=pl.BlockSpec((br, _D), lambda j: (j, 0)),
        out_shape=jax.ShapeDtypeStruct((_M, _D), jnp.float32),
    )(outt)


def kernel(x, dim, index, src, include_self):
    xt = _tc_transpose_x(x).reshape(-1)
    srct, idxt = _tc_transpose_src_idx(src, index.astype(jnp.int32))
    outt = _make_sc_scatter()(xt, idxt.reshape(-1), srct.reshape(-1))
    return _tc_transpose_out(outt.reshape(_D, _MP // _D, _D))


# speedup vs baseline: 2.2322x; 1.0818x over previous
import functools

import jax
import jax.numpy as jnp
from jax import lax
from jax.experimental import pallas as pl
from jax.experimental.pallas import tpu as pltpu
from jax.experimental.pallas import tpu_sc as plsc

_M = 100000
_MP = 100352
_B = 16384
_D = 128
_NT = 16
_NC = 2
_ROUNDS = _D // (_NT * _NC)
_L = 16
_CH = 8192
_H0, _H1 = 49152, _MP - 49152


def _make_sc_scatter():
    mesh = plsc.VectorSubcoreMesh(core_axis_name="c", subcore_axis_name="s")

    @functools.partial(
        pl.kernel,
        mesh=mesh,
        out_type=jax.ShapeDtypeStruct((_D * _MP,), jnp.float32),
        scratch_types=[
            pltpu.VMEM_SHARED((_NT * _MP,), jnp.float32),
            pltpu.VMEM((_CH,), jnp.int32),
            pltpu.VMEM((_CH,), jnp.float32),
            pltpu.SemaphoreType.DMA,
            pltpu.SemaphoreType.DMA,
        ],
    )
    def sc_scatter(xt_hbm, idxt_hbm, srct_hbm, outt_hbm,
                   acc_sh, idx_v, src_v, sem_x, sem_w):
        c = lax.axis_index("c")
        s = lax.axis_index("s")
        base = s * _MP

        def round_body(r, carry):
            col = c * (_ROUNDS * _NT) + r * _NT + s

            @pl.when(r > 0)
            def _():
                for off, ln in ((0, _H0), (_H0, _H1)):
                    pltpu.make_async_copy(
                        acc_sh.at[pl.ds(base + off, ln)],
                        outt_hbm.at[pl.ds(col * _MP + off, ln)],
                        sem_w).wait()

            cx = [pltpu.async_copy(xt_hbm.at[pl.ds(col * _MP + off, ln)],
                                   acc_sh.at[pl.ds(base + off, ln)], sem_x)
                  for off, ln in ((0, _H0), (_H0, _H1))]

            def chunk_body(h, carry2):
                cbase = col * _B + h * _CH
                pltpu.sync_copy(idxt_hbm.at[pl.ds(cbase, _CH)], idx_v)
                pltpu.sync_copy(srct_hbm.at[pl.ds(cbase, _CH)], src_v)

                @pl.when(h == 0)
                def _():
                    for c_ in cx:
                        c_.wait()

                pltpu.sync_copy(src_v, acc_sh.at[idx_v], add=True)
                return carry2

            lax.fori_loop(0, _B // _CH, chunk_body, 0)
            for off, ln in ((0, _H0), (_H0, _H1)):
                pltpu.async_copy(acc_sh.at[pl.ds(base + off, ln)],
                                 outt_hbm.at[pl.ds(col * _MP + off, ln)],
                                 sem_w)
            return carry

        lax.fori_loop(0, _ROUNDS, round_body, 0)
        last = (_ROUNDS - 1) * _NT + _NT - 1
        for off, ln in ((0, _H0), (_H0, _H1)):
            pltpu.make_async_copy(
                acc_sh.at[pl.ds(base + off, ln)],
                outt_hbm.at[pl.ds(last * _MP + off, ln)],
                sem_w).wait()

    return sc_scatter


def _tc_transpose_x(x):
    br = 14336

    def body(x_ref, o_ref):
        o_ref[...] = x_ref[...].T.reshape(_D, br // _D, _D)

    return pl.pallas_call(
        body,
        grid=(_MP // br,),
        in_specs=[pl.BlockSpec((br, _D), lambda j: (j, 0))],
        out_specs=pl.BlockSpec((_D, br // _D, _D), lambda j: (0, j, 0)),
        out_shape=jax.ShapeDtypeStruct((_D, _MP // _D, _D), jnp.float32),
    )(x)


def _tc_transpose_src_idx(src, idx):
    br = 8192

    def body(s_ref, i_ref, os_ref, oi_ref):
        cmap = (lax.broadcasted_iota(jnp.int32, (_D, 1), 0) % _NT) * _MP
        os_ref[...] = s_ref[...].T.reshape(_D, br // _D, _D)
        oi_ref[...] = (i_ref[...].T + cmap).reshape(_D, br // _D, _D)

    return pl.pallas_call(
        body,
        grid=(_B // br,),
        in_specs=[pl.BlockSpec((br, _D), lambda j: (j, 0)),
                  pl.BlockSpec((br, _D), lambda j: (j, 0))],
        out_specs=[pl.BlockSpec((_D, br // _D, _D), lambda j: (0, j, 0)),
                   pl.BlockSpec((_D, br // _D, _D), lambda j: (0, j, 0))],
        out_shape=[jax.ShapeDtypeStruct((_D, _B // _D, _D), jnp.float32),
                   jax.ShapeDtypeStruct((_D, _B // _D, _D), jnp.int32)],
    )(src, idx)


def _tc_transpose_out(outt):
    br = 14336

    def body(t_ref, o_ref):
        o_ref[...] = t_ref[...].reshape(_D, br).T

    return pl.pallas_call(
        body,
        grid=(_MP // br,),
        in_specs=[pl.BlockSpec((_D, br // _D, _D), lambda j: (0, j, 0))],
        out_specs=pl.BlockSpec((br, _D), lambda j: (j, 0)),
        out_shape=jax.ShapeDtypeStruct((_M, _D), jnp.float32),
    )(outt)


def kernel(x, dim, index, src, include_self):
    xt = _tc_transpose_x(x).reshape(-1)
    srct, idxt = _tc_transpose_src_idx(src, index.astype(jnp.int32))
    outt = _make_sc_scatter()(xt, idxt.reshape(-1), srct.reshape(-1))
    return _tc_transpose_out(outt.reshape(_D, _MP // _D, _D))
